# Initial kernel scaffold; baseline (speedup 1.0000x reference)
import functools

import jax
import jax.numpy as jnp
from jax import lax
from jax.experimental import pallas as pl
from jax.experimental.pallas import tpu as pltpu
from jax.experimental.pallas import tpu_sc as plsc

B = 4096
T = 50
D = 32
E = 2 * D


NC, NS = 2, 16
NW = NC * NS
HIST_PER_W = (B * T) // NW
HCHUNK = 1280
SMALL_PER_W = B // NW


def _sc_gather(emb_user, emb_item, emb_cate, idx_user, idx_item, idx_cate,
               idx_hi, idx_hc):
    mesh = plsc.VectorSubcoreMesh(core_axis_name="c", subcore_axis_name="s",
                                  num_cores=NC, num_subcores=NS)
    out_type = (
        jax.ShapeDtypeStruct((B * T, D), jnp.float32),
        jax.ShapeDtypeStruct((B * T, D), jnp.float32),
        jax.ShapeDtypeStruct((B, D), jnp.float32),
        jax.ShapeDtypeStruct((B, D), jnp.float32),
        jax.ShapeDtypeStruct((B, D), jnp.float32),
    )
    scratch = [
        pltpu.VMEM((HIST_PER_W,), jnp.int32),
        pltpu.VMEM((HCHUNK, D), jnp.float32),
        pltpu.VMEM((SMALL_PER_W,), jnp.int32),
        pltpu.VMEM((SMALL_PER_W, D), jnp.float32),
        pltpu.SemaphoreType.DMA,
    ]

    @functools.partial(pl.kernel, out_type=out_type, mesh=mesh,
                       scratch_types=scratch)
    def k(eu, ei, ec, iu, ii, ic, ihi, ihc,
          o_hi, o_hc, o_u, o_i, o_c, hidx_v, rows_v, sidx_v, srows_v, sem):
        wid = lax.axis_index("s") * NC + lax.axis_index("c")
        hbase = wid * HIST_PER_W
        for idx_hbm, tab, out_hbm in ((ihi, ei, o_hi), (ihc, ec, o_hc)):
            pltpu.sync_copy(idx_hbm.at[pl.ds(hbase, HIST_PER_W)], hidx_v)
            for c in range(HIST_PER_W // HCHUNK):
                pltpu.async_copy(
                    tab.at[hidx_v.at[pl.ds(c * HCHUNK, HCHUNK)]], rows_v,
                    sem).wait()
                pltpu.sync_copy(rows_v,
                                out_hbm.at[pl.ds(hbase + c * HCHUNK, HCHUNK)])
        sbase = wid * SMALL_PER_W
        for idx_hbm, tab, out_hbm in ((iu, eu, o_u), (ii, ei, o_i),
                                      (ic, ec, o_c)):
            pltpu.sync_copy(idx_hbm.at[pl.ds(sbase, SMALL_PER_W)], sidx_v)
            pltpu.async_copy(tab.at[sidx_v], srows_v, sem).wait()
            pltpu.sync_copy(srows_v, out_hbm.at[pl.ds(sbase, SMALL_PER_W)])

    return k(emb_user, emb_item, emb_cate, idx_user, idx_item, idx_cate,
             idx_hi, idx_hc)


BB = 512
NB = B // BB
CH = 10
NCH = T // CH


def _tc_body(hi_ref, hc_ref, ue_ref, ie_ref, ce_ref, len_ref,
             gWihT, gWhhT, g_bih, g_bhh,
             aW_q, aW_i, aW_m, a_b1, aW2, a_b2, aWd, a_bd,
             xWihT, xWhhT, x_bih, x_bhh,
             dW1, d_b1, dW2, d_b2, oW, o_b,
             out_ref, gi_ref, intr_ref, att_ref):
    f32 = jnp.float32
    lens = len_ref[...]

    for c in range(NCH):
        kc = jnp.concatenate([hi_ref[c * CH:(c + 1) * CH],
                              hc_ref[c * CH:(c + 1) * CH]], axis=2)
        gic = jnp.dot(kc.reshape(CH * BB, E), gWihT[...],
                      preferred_element_type=f32) + g_bih[...]
        gi_ref[c * CH:(c + 1) * CH] = gic.reshape(CH, BB, 3 * E)

    def gru_step(t, h):
        git = gi_ref[t]
        gh = jnp.dot(h, gWhhT[...], preferred_element_type=f32) + g_bhh[...]
        rz = jax.nn.sigmoid(git[:, :2 * E] + gh[:, :2 * E])
        n = jnp.tanh(git[:, 2 * E:] + rz[:, :E] * gh[:, 2 * E:])
        h_new = (1.0 - rz[:, E:]) * n + rz[:, E:] * h
        intr_ref[t] = h_new
        return h_new

    lax.fori_loop(0, T, gru_step, jnp.zeros((BB, E), f32))

    q = jnp.concatenate([ie_ref[...], ce_ref[...]], axis=1)
    qc = jnp.dot(q, aW_q[...], preferred_element_type=f32) + a_b1[...]
    for c in range(NCH):
        ic3 = intr_ref[c * CH:(c + 1) * CH]
        im = (ic3 * q[None, :, :]).reshape(CH * BB, E)
        x = (jnp.dot(ic3.reshape(CH * BB, E), aW_i[...],
                     preferred_element_type=f32)
             + jnp.dot(im, aW_m[...], preferred_element_type=f32))
        h1 = jax.nn.relu(x.reshape(CH, BB, E) + qc[None, :, :])
        h2 = jax.nn.relu(jnp.dot(h1.reshape(CH * BB, E), aW2[...],
                                 preferred_element_type=f32) + a_b2[...])
        sc = jnp.dot(h2, aWd[...], preferred_element_type=f32) + a_bd[...]
        att_ref[c * CH:(c + 1) * CH] = sc.reshape(CH, BB, 1)

    sc = att_ref[...]
    tidx = lax.broadcasted_iota(jnp.int32, (T, BB, 1), 0)
    sc = jnp.where(tidx < lens[None, :, :], sc, -(2.0 ** 32) + 1.0)
    e = jnp.exp(sc - jnp.max(sc, axis=0, keepdims=True))
    att_ref[...] = e / jnp.sum(e, axis=0, keepdims=True)

    for c in range(NCH):
        ic3 = intr_ref[c * CH:(c + 1) * CH]
        gic = jnp.dot(ic3.reshape(CH * BB, E), xWihT[...],
                      preferred_element_type=f32) + x_bih[...]
        gi_ref[c * CH:(c + 1) * CH] = gic.reshape(CH, BB, 3 * E)

    def aug_step(t, h):
        git = gi_ref[t]
        gh = jnp.dot(h, xWhhT[...], preferred_element_type=f32) + x_bhh[...]
        ru = jax.nn.sigmoid(git[:, :2 * E] + gh[:, :2 * E])
        n = jnp.tanh(git[:, 2 * E:] + ru[:, :E] * gh[:, 2 * E:])
        u2 = att_ref[t] * ru[:, E:]
        h_new = (1.0 - u2) * h + u2 * n
        return jnp.where(t < lens, h_new, h)

    hT = lax.fori_loop(0, T, aug_step, jnp.zeros((BB, E), f32))

    d1 = jax.nn.relu(
        jnp.dot(hT, dW1[0:E], preferred_element_type=f32)
        + jnp.dot(ue_ref[...], dW1[E:E + D], preferred_element_type=f32)
        + jnp.dot(ie_ref[...], dW1[E + D:E + 2 * D],
                  preferred_element_type=f32)
        + jnp.dot(ce_ref[...], dW1[E + 2 * D:E + 3 * D],
                  preferred_element_type=f32)
        + d_b1[...])
    d2 = jax.nn.relu(jnp.dot(d1, dW2[...], preferred_element_type=f32)
                     + d_b2[...])
    logit = jnp.dot(d2, oW[...], preferred_element_type=f32) + o_b[...]
    out_ref[...] = jax.nn.sigmoid(logit)


def _tc_forward(hi_T, hc_T, user_e, item_e, cate_e, lens2, *weights):
    def full(w):
        nd = w.ndim
        return pl.BlockSpec(w.shape, lambda i, _n=nd: (0,) * _n)

    in_specs = [
        pl.BlockSpec((T, BB, D), lambda i: (0, i, 0)),
        pl.BlockSpec((T, BB, D), lambda i: (0, i, 0)),
        pl.BlockSpec((BB, D), lambda i: (i, 0)),
        pl.BlockSpec((BB, D), lambda i: (i, 0)),
        pl.BlockSpec((BB, D), lambda i: (i, 0)),
        pl.BlockSpec((BB, 1), lambda i: (i, 0)),
    ] + [full(w) for w in weights]
    return pl.pallas_call(
        _tc_body,
        grid=(NB,),
        in_specs=in_specs,
        out_specs=pl.BlockSpec((BB, 1), lambda i: (i, 0)),
        out_shape=jax.ShapeDtypeStruct((B, 1), jnp.float32),
        scratch_shapes=[
            pltpu.VMEM((T, BB, 3 * E), jnp.float32),
            pltpu.VMEM((T, BB, E), jnp.float32),
            pltpu.VMEM((T, BB, 1), jnp.float32),
        ],
    )(hi_T, hc_T, user_e, item_e, cate_e, lens2, *weights)


def kernel(X, emb_user, emb_item, emb_cate, gru_Wih, gru_Whh, gru_bih,
           gru_bhh, att_W1, att_b1, att_W2, att_b2, att_Wd, att_bd,
           aug_Wih, aug_Whh, aug_bih, aug_bhh, dnn_W1, dnn_b1, dnn_W2,
           dnn_b2, out_W, out_b, pred_bias):
    Xi = X.astype(jnp.int32)
    idx_user = Xi[:, 0]
    idx_item = Xi[:, 1]
    idx_cate = Xi[:, 2]
    idx_hi = Xi[:, 3:3 + T].T.reshape(B * T)
    idx_hc = Xi[:, 3 + T:3 + 2 * T].T.reshape(B * T)
    lens2 = Xi[:, 3 + 2 * T:3 + 2 * T + 1]

    hi_e, hc_e, user_e, item_e, cate_e = _sc_gather(
        emb_user, emb_item, emb_cate, idx_user, idx_item, idx_cate,
        idx_hi, idx_hc)
    hi_T = hi_e.reshape(T, B, D)
    hc_T = hc_e.reshape(T, B, D)

    weights = (
        gru_Wih.T, gru_Whh.T,
        gru_bih.reshape(1, 3 * E), gru_bhh.reshape(1, 3 * E),
        att_W1[0:E] + att_W1[2 * E:3 * E],
        att_W1[E:2 * E] - att_W1[2 * E:3 * E],
        att_W1[3 * E:4 * E],
        att_b1.reshape(1, E), att_W2, att_b2.reshape(1, 16),
        att_Wd, att_bd.reshape(1, 1),
        aug_Wih.T, aug_Whh.T,
        aug_bih.reshape(1, 3 * E), aug_bhh.reshape(1, 3 * E),
        dnn_W1, dnn_b1.reshape(1, 256), dnn_W2, dnn_b2.reshape(1, 128),
        out_W, (out_b + pred_bias).reshape(1, 1),
    )
    return _tc_forward(hi_T, hc_T, user_e, item_e, cate_e, lens2, *weights)


# speedup vs baseline: 2.4595x; 2.4595x over previous
import functools

import jax
import jax.numpy as jnp
from jax import lax
from jax.experimental import pallas as pl
from jax.experimental.pallas import tpu as pltpu
from jax.experimental.pallas import tpu_sc as plsc

B = 4096
T = 50
D = 32
E = 2 * D
BH2 = B // 2


NC, NS = 2, 16
NW = NC * NS
KEYROWS = T * BH2
KPW = KEYROWS // NW
KCH = 800
QPW = BH2 // NW


def _sc_gather(emb_user, emb_item, emb_cate, ki, qi, ui):
    mesh = plsc.VectorSubcoreMesh(core_axis_name="c", subcore_axis_name="s",
                                  num_cores=NC, num_subcores=NS)
    out_type = (
        jax.ShapeDtypeStruct((KEYROWS, 4 * D), jnp.float32),
        jax.ShapeDtypeStruct((BH2, 4 * D), jnp.float32),
        jax.ShapeDtypeStruct((BH2, 2 * D), jnp.float32),
    )
    scratch = [
        pltpu.VMEM((KPW,), jnp.int32),
        pltpu.VMEM((KCH, D), jnp.float32),
        pltpu.VMEM((QPW,), jnp.int32),
        pltpu.VMEM((QPW, D), jnp.float32),
        pltpu.SemaphoreType.DMA,
    ]

    @functools.partial(
        pl.kernel, out_type=out_type, mesh=mesh, scratch_types=scratch,
        compiler_params=pltpu.CompilerParams(use_tc_tiling_on_sc=False))
    def k(eu, ei, ec, ki_h, qi_h, ui_h, o_k, o_q, o_u,
          kidx_v, rows_v, sidx_v, srows_v, sem):
        wid = lax.axis_index("s") * NC + lax.axis_index("c")
        tabs = (ei, ec, ei, ec)
        for g in range(4):
            pltpu.sync_copy(ki_h.at[pl.ds(g * KEYROWS + wid * KPW, KPW)],
                            kidx_v)
            for c in range(KPW // KCH):
                pltpu.async_copy(
                    tabs[g].at[kidx_v.at[pl.ds(c * KCH, KCH)]], rows_v,
                    sem).wait()
                pltpu.sync_copy(
                    rows_v,
                    o_k.at[pl.ds(wid * KPW + c * KCH, KCH),
                           pl.ds(g * D, D)])
        for g in range(4):
            pltpu.sync_copy(qi_h.at[pl.ds(g * BH2 + wid * QPW, QPW)], sidx_v)
            pltpu.async_copy(tabs[g].at[sidx_v], srows_v, sem).wait()
            pltpu.sync_copy(srows_v,
                            o_q.at[pl.ds(wid * QPW, QPW), pl.ds(g * D, D)])
        for g in range(2):
            pltpu.sync_copy(ui_h.at[pl.ds(g * BH2 + wid * QPW, QPW)], sidx_v)
            pltpu.async_copy(eu.at[sidx_v], srows_v, sem).wait()
            pltpu.sync_copy(srows_v,
                            o_u.at[pl.ds(wid * QPW, QPW), pl.ds(g * D, D)])

    return k(emb_user, emb_item, emb_cate, ki, qi, ui)


BH = 256
NB = BH2 // BH
CH = 10
NCH = T // CH


def _tc_body(keys_ref, qw_ref, uw_ref, len_ref,
             gWih_b, gWhh_b, g_bih, g_bhh,
             aWq_b, aWi_b, aWm_b, a_b1, aW2_b, a_b2, aWd_b, a_bd,
             xWih_b, xWhh_b, x_bih, x_bhh,
             dW1, d_b1, dW2, d_b2, oW, o_b,
             out_ref, gi_ref, intr_ref):
    f32 = jnp.float32
    lens_w = len_ref[...]

    for c in range(NCH):
        kb = keys_ref[c * CH:(c + 1) * CH]
        gw = jnp.dot(kb.reshape(CH * BH, 2 * E), gWih_b[...],
                     preferred_element_type=f32) + g_bih[...]
        gi_ref[c * CH:(c + 1) * CH, :, 0:6 * E] = gw.reshape(CH, BH, 6 * E)

    def gru_step(t, hw):
        git = gi_ref[t]
        ghw = jnp.dot(hw, gWhh_b[...],
                      preferred_element_type=f32) + g_bhh[...]
        hs = []
        for k in range(2):
            g0 = 3 * E * k
            rz = jax.nn.sigmoid(git[:, g0:g0 + 2 * E]
                                + ghw[:, g0:g0 + 2 * E])
            n = jnp.tanh(git[:, g0 + 2 * E:g0 + 3 * E]
                         + rz[:, 0:E] * ghw[:, g0 + 2 * E:g0 + 3 * E])
            z = rz[:, E:2 * E]
            hk = hw[:, E * k:E * k + E]
            hs.append((1.0 - z) * n + z * hk)
        hn = jnp.concatenate(hs, axis=1)
        intr_ref[t] = hn
        return hn

    lax.fori_loop(0, T, gru_step, jnp.zeros((BH, 2 * E), f32))

    qw = qw_ref[...]
    qc = jnp.dot(qw, aWq_b[...], preferred_element_type=f32) + a_b1[...]
    for c in range(NCH):
        ic = intr_ref[c * CH:(c + 1) * CH]
        imw = (ic * qw[None, :, :]).reshape(CH * BH, 2 * E)
        xw = (jnp.dot(ic.reshape(CH * BH, 2 * E), aWi_b[...],
                      preferred_element_type=f32)
              + jnp.dot(imw, aWm_b[...], preferred_element_type=f32))
        h1 = jax.nn.relu(xw.reshape(CH, BH, 2 * E) + qc[None, :, :])
        h2 = jax.nn.relu(jnp.dot(h1.reshape(CH * BH, 2 * E), aW2_b[...],
                                 preferred_element_type=f32) + a_b2[...])
        scw = jnp.dot(h2, aWd_b[...], preferred_element_type=f32) + a_bd[...]
        gi_ref[c * CH:(c + 1) * CH, :, 6 * E:6 * E + 2] = \
            scw.reshape(CH, BH, 2)

    acc = jnp.zeros((BH, 2), f32)
    for c in range(NCH):
        scc = gi_ref[c * CH:(c + 1) * CH, :, 6 * E:6 * E + 2]
        tidx = c * CH + lax.broadcasted_iota(jnp.int32, (CH, BH, 2), 0)
        ec = jnp.where(tidx < lens_w[None, :, :], jnp.exp(scc), 0.0)
        gi_ref[c * CH:(c + 1) * CH, :, 6 * E:6 * E + 2] = ec
        acc = acc + jnp.sum(ec, axis=0)
    inv = 1.0 / acc
    for c in range(NCH):
        ec = gi_ref[c * CH:(c + 1) * CH, :, 6 * E:6 * E + 2]
        gi_ref[c * CH:(c + 1) * CH, :, 6 * E:6 * E + 2] = \
            ec * inv[None, :, :]

    for c in range(NCH):
        ic = intr_ref[c * CH:(c + 1) * CH]
        gw = jnp.dot(ic.reshape(CH * BH, 2 * E), xWih_b[...],
                     preferred_element_type=f32) + x_bih[...]
        gi_ref[c * CH:(c + 1) * CH, :, 0:6 * E] = gw.reshape(CH, BH, 6 * E)

    def aug_step(t, hw):
        git = gi_ref[t]
        ghw = jnp.dot(hw, xWhh_b[...],
                      preferred_element_type=f32) + x_bhh[...]
        hs = []
        for k in range(2):
            g0 = 3 * E * k
            rz = jax.nn.sigmoid(git[:, g0:g0 + 2 * E]
                                + ghw[:, g0:g0 + 2 * E])
            n = jnp.tanh(git[:, g0 + 2 * E:g0 + 3 * E]
                         + rz[:, 0:E] * ghw[:, g0 + 2 * E:g0 + 3 * E])
            u2 = git[:, 6 * E + k:6 * E + k + 1] * rz[:, E:2 * E]
            hk = hw[:, E * k:E * k + E]
            hnk = (1.0 - u2) * hk + u2 * n
            hs.append(jnp.where(t < lens_w[:, k:k + 1], hnk, hk))
        return jnp.concatenate(hs, axis=1)

    hTw = lax.fori_loop(0, T, aug_step, jnp.zeros((BH, 2 * E), f32))

    for k in range(2):
        hk = hTw[:, E * k:E * k + E]
        uk = uw_ref[:, D * k:D * k + D]
        ik = qw[:, 2 * D * k:2 * D * k + D]
        ck = qw[:, 2 * D * k + D:2 * D * k + 2 * D]
        d1 = jax.nn.relu(
            jnp.dot(hk, dW1[0:E], preferred_element_type=f32)
            + jnp.dot(uk, dW1[E:E + D], preferred_element_type=f32)
            + jnp.dot(ik, dW1[E + D:E + 2 * D], preferred_element_type=f32)
            + jnp.dot(ck, dW1[E + 2 * D:E + 3 * D],
                      preferred_element_type=f32)
            + d_b1[...])
        d2 = jax.nn.relu(jnp.dot(d1, dW2[...], preferred_element_type=f32)
                         + d_b2[...])
        logit = jnp.dot(d2, oW[...], preferred_element_type=f32) + o_b[...]
        out_ref[k] = jax.nn.sigmoid(logit)


def _tc_forward(keys3, qw, uw, lens_w, *weights):
    def full(w):
        nd = w.ndim
        return pl.BlockSpec(w.shape, lambda i, _n=nd: (0,) * _n)

    in_specs = [
        pl.BlockSpec((T, BH, 4 * D), lambda i: (0, i, 0)),
        pl.BlockSpec((BH, 4 * D), lambda i: (i, 0)),
        pl.BlockSpec((BH, 2 * D), lambda i: (i, 0)),
        pl.BlockSpec((BH, 2), lambda i: (i, 0)),
    ] + [full(w) for w in weights]
    return pl.pallas_call(
        _tc_body,
        grid=(NB,),
        in_specs=in_specs,
        out_specs=pl.BlockSpec((2, BH, 1), lambda i: (0, i, 0)),
        out_shape=jax.ShapeDtypeStruct((2, BH2, 1), jnp.float32),
        scratch_shapes=[
            pltpu.VMEM((T, BH, 8 * E), jnp.float32),
            pltpu.VMEM((T, BH, 2 * E), jnp.float32),
        ],
    )(keys3, qw, uw, lens_w, *weights)


def _bdiag(w):
    z = jnp.zeros_like(w)
    return jnp.concatenate([jnp.concatenate([w, z], 1),
                            jnp.concatenate([z, w], 1)], 0)


def _t2(b):
    return jnp.concatenate([b, b]).reshape(1, -1)


def kernel(X, emb_user, emb_item, emb_cate, gru_Wih, gru_Whh, gru_bih,
           gru_bhh, att_W1, att_b1, att_W2, att_b2, att_Wd, att_bd,
           aug_Wih, aug_Whh, aug_bih, aug_bhh, dnn_W1, dnn_b1, dnn_W2,
           dnn_b2, out_W, out_b, pred_bias):
    Xi = X.astype(jnp.int32)
    hiT = Xi[:, 3:3 + T].T
    hcT = Xi[:, 3 + T:3 + 2 * T].T
    ki = jnp.concatenate([hiT[:, :BH2].reshape(-1), hcT[:, :BH2].reshape(-1),
                          hiT[:, BH2:].reshape(-1), hcT[:, BH2:].reshape(-1)])
    qi = jnp.concatenate([Xi[:BH2, 1], Xi[:BH2, 2],
                          Xi[BH2:, 1], Xi[BH2:, 2]])
    ui = Xi[:, 0]
    lens_w = jnp.stack([Xi[:BH2, 3 + 2 * T], Xi[BH2:, 3 + 2 * T]], axis=1)

    keysw, qw, uw = _sc_gather(emb_user, emb_item, emb_cate, ki, qi, ui)
    keys3 = keysw.reshape(T, BH2, 4 * D)

    aW_q = att_W1[0:E] + att_W1[2 * E:3 * E]
    aW_i = att_W1[E:2 * E] - att_W1[2 * E:3 * E]
    aW_m = att_W1[3 * E:4 * E]
    weights = (
        _bdiag(gru_Wih.T), _bdiag(gru_Whh.T),
        _t2(gru_bih), _t2(gru_bhh),
        _bdiag(aW_q), _bdiag(aW_i), _bdiag(aW_m),
        _t2(att_b1), _bdiag(att_W2), _t2(att_b2),
        _bdiag(att_Wd), _t2(att_bd),
        _bdiag(aug_Wih.T), _bdiag(aug_Whh.T),
        _t2(aug_bih), _t2(aug_bhh),
        dnn_W1, dnn_b1.reshape(1, 256), dnn_W2, dnn_b2.reshape(1, 128),
        out_W, (out_b + pred_bias).reshape(1, 1),
    )
    out2 = _tc_forward(keys3, qw, uw, lens_w, *weights)
    return out2.reshape(B, 1)


# speedup vs baseline: 2.4618x; 1.0009x over previous
import functools

import jax
import jax.numpy as jnp
from jax import lax
from jax.experimental import pallas as pl
from jax.experimental.pallas import tpu as pltpu
from jax.experimental.pallas import tpu_sc as plsc

B = 4096
T = 50
D = 32
E = 2 * D
BH2 = B // 2


NC, NS = 2, 16
NW = NC * NS
KEYROWS = T * BH2
KPW = KEYROWS // NW
KCH = 800
QPW = BH2 // NW


def _sc_gather(emb_user, emb_item, emb_cate, ki, qi, ui):
    mesh = plsc.VectorSubcoreMesh(core_axis_name="c", subcore_axis_name="s",
                                  num_cores=NC, num_subcores=NS)
    out_type = (
        jax.ShapeDtypeStruct((KEYROWS, 4 * D), jnp.float32),
        jax.ShapeDtypeStruct((BH2, 4 * D), jnp.float32),
        jax.ShapeDtypeStruct((BH2, 4 * D), jnp.float32),
    )
    scratch = [
        pltpu.VMEM((KPW,), jnp.int32),
        pltpu.VMEM((KCH, D), jnp.float32),
        pltpu.VMEM((QPW,), jnp.int32),
        pltpu.VMEM((QPW, D), jnp.float32),
        pltpu.SemaphoreType.DMA,
    ]

    @functools.partial(
        pl.kernel, out_type=out_type, mesh=mesh, scratch_types=scratch,
        compiler_params=pltpu.CompilerParams(use_tc_tiling_on_sc=False))
    def k(eu, ei, ec, ki_h, qi_h, ui_h, o_k, o_q, o_u,
          kidx_v, rows_v, sidx_v, srows_v, sem):
        wid = lax.axis_index("s") * NC + lax.axis_index("c")
        tabs = (ei, ec, ei, ec)
        for g in range(4):
            pltpu.sync_copy(ki_h.at[pl.ds(g * KEYROWS + wid * KPW, KPW)],
                            kidx_v)
            for c in range(KPW // KCH):
                pltpu.async_copy(
                    tabs[g].at[kidx_v.at[pl.ds(c * KCH, KCH)]], rows_v,
                    sem).wait()
                pltpu.sync_copy(
                    rows_v,
                    o_k.at[pl.ds(wid * KPW + c * KCH, KCH),
                           pl.ds(g * D, D)])
        for g in range(4):
            pltpu.sync_copy(qi_h.at[pl.ds(g * BH2 + wid * QPW, QPW)], sidx_v)
            pltpu.async_copy(tabs[g].at[sidx_v], srows_v, sem).wait()
            pltpu.sync_copy(srows_v,
                            o_q.at[pl.ds(wid * QPW, QPW), pl.ds(g * D, D)])
        for g in range(2):
            pltpu.sync_copy(ui_h.at[pl.ds(g * BH2 + wid * QPW, QPW)], sidx_v)
            pltpu.async_copy(eu.at[sidx_v], srows_v, sem).wait()
            pltpu.sync_copy(srows_v,
                            o_u.at[pl.ds(wid * QPW, QPW), pl.ds(g * D, D)])

    return k(emb_user, emb_item, emb_cate, ki, qi, ui)


BH = 256
NB = BH2 // BH
CH = 10
NCH = T // CH


def _tc_body(keys_ref, qw_ref, uw_ref, len_ref,
             gWih_b, gWhh_b, g_bih, g_bhh,
             aWq_b, aWi_b, aWm_b, a_b1, aW2_b, a_b2, aWd_b, a_bd,
             xWih_b, xWhh_b, x_bih, x_bhh,
             dW1, d_b1, dW2, d_b2, oW, o_b,
             out_ref, gi_ref, intr_ref):
    f32 = jnp.float32
    lens_w = len_ref[...]

    for c in range(NCH):
        kb = keys_ref[pl.ds(c * CH * BH, CH * BH)]
        gw = jnp.dot(kb, gWih_b[...],
                     preferred_element_type=f32) + g_bih[...]
        gi_ref[c * CH:(c + 1) * CH, :, 0:6 * E] = gw.reshape(CH, BH, 6 * E)

    def gru_step(t, hw):
        git = gi_ref[t]
        ghw = jnp.dot(hw, gWhh_b[...],
                      preferred_element_type=f32) + g_bhh[...]
        hs = []
        for k in range(2):
            g0 = 3 * E * k
            rz = jax.nn.sigmoid(git[:, g0:g0 + 2 * E]
                                + ghw[:, g0:g0 + 2 * E])
            n = jnp.tanh(git[:, g0 + 2 * E:g0 + 3 * E]
                         + rz[:, 0:E] * ghw[:, g0 + 2 * E:g0 + 3 * E])
            z = rz[:, E:2 * E]
            hk = hw[:, E * k:E * k + E]
            hs.append((1.0 - z) * n + z * hk)
        hn = jnp.concatenate(hs, axis=1)
        intr_ref[t] = hn
        return hn

    lax.fori_loop(0, T, gru_step, jnp.zeros((BH, 2 * E), f32))

    qw = qw_ref[...]
    qc = jnp.dot(qw, aWq_b[...], preferred_element_type=f32) + a_b1[...]
    for c in range(NCH):
        ic = intr_ref[c * CH:(c + 1) * CH]
        imw = (ic * qw[None, :, :]).reshape(CH * BH, 2 * E)
        xw = (jnp.dot(ic.reshape(CH * BH, 2 * E), aWi_b[...],
                      preferred_element_type=f32)
              + jnp.dot(imw, aWm_b[...], preferred_element_type=f32))
        h1 = jax.nn.relu(xw.reshape(CH, BH, 2 * E) + qc[None, :, :])
        h2 = jax.nn.relu(jnp.dot(h1.reshape(CH * BH, 2 * E), aW2_b[...],
                                 preferred_element_type=f32) + a_b2[...])
        scw = jnp.dot(h2, aWd_b[...], preferred_element_type=f32) + a_bd[...]
        gi_ref[c * CH:(c + 1) * CH, :, 6 * E:6 * E + 2] = \
            scw.reshape(CH, BH, 2)

    acc = jnp.zeros((BH, 2), f32)
    for c in range(NCH):
        scc = gi_ref[c * CH:(c + 1) * CH, :, 6 * E:6 * E + 2]
        tidx = c * CH + lax.broadcasted_iota(jnp.int32, (CH, BH, 2), 0)
        ec = jnp.where(tidx < lens_w[None, :, :], jnp.exp(scc), 0.0)
        gi_ref[c * CH:(c + 1) * CH, :, 6 * E:6 * E + 2] = ec
        acc = acc + jnp.sum(ec, axis=0)
    inv = 1.0 / acc
    for c in range(NCH):
        ec = gi_ref[c * CH:(c + 1) * CH, :, 6 * E:6 * E + 2]
        gi_ref[c * CH:(c + 1) * CH, :, 6 * E:6 * E + 2] = \
            ec * inv[None, :, :]

    for c in range(NCH):
        ic = intr_ref[c * CH:(c + 1) * CH]
        gw = jnp.dot(ic.reshape(CH * BH, 2 * E), xWih_b[...],
                     preferred_element_type=f32) + x_bih[...]
        gi_ref[c * CH:(c + 1) * CH, :, 0:6 * E] = gw.reshape(CH, BH, 6 * E)

    def aug_step(t, hw):
        git = gi_ref[t]
        ghw = jnp.dot(hw, xWhh_b[...],
                      preferred_element_type=f32) + x_bhh[...]
        hs = []
        for k in range(2):
            g0 = 3 * E * k
            rz = jax.nn.sigmoid(git[:, g0:g0 + 2 * E]
                                + ghw[:, g0:g0 + 2 * E])
            n = jnp.tanh(git[:, g0 + 2 * E:g0 + 3 * E]
                         + rz[:, 0:E] * ghw[:, g0 + 2 * E:g0 + 3 * E])
            u2 = git[:, 6 * E + k:6 * E + k + 1] * rz[:, E:2 * E]
            hk = hw[:, E * k:E * k + E]
            hnk = (1.0 - u2) * hk + u2 * n
            hs.append(jnp.where(t < lens_w[:, k:k + 1], hnk, hk))
        return jnp.concatenate(hs, axis=1)

    hTw = lax.fori_loop(0, T, aug_step, jnp.zeros((BH, 2 * E), f32))

    for k in range(2):
        hk = hTw[:, E * k:E * k + E]
        uk = uw_ref[:, D * k:D * k + D]
        ik = qw[:, 2 * D * k:2 * D * k + D]
        ck = qw[:, 2 * D * k + D:2 * D * k + 2 * D]
        d1 = jax.nn.relu(
            jnp.dot(hk, dW1[0:E], preferred_element_type=f32)
            + jnp.dot(uk, dW1[E:E + D], preferred_element_type=f32)
            + jnp.dot(ik, dW1[E + D:E + 2 * D], preferred_element_type=f32)
            + jnp.dot(ck, dW1[E + 2 * D:E + 3 * D],
                      preferred_element_type=f32)
            + d_b1[...])
        d2 = jax.nn.relu(jnp.dot(d1, dW2[...], preferred_element_type=f32)
                         + d_b2[...])
        logit = jnp.dot(d2, oW[...], preferred_element_type=f32) + o_b[...]
        out_ref[k] = jax.nn.sigmoid(logit)


def _tc_forward(keys3, qw, uw, lens_w, *weights):
    def full(w):
        nd = w.ndim
        return pl.BlockSpec(w.shape, lambda i, _n=nd: (0,) * _n)

    in_specs = [
        pl.BlockSpec((T * BH, 4 * D), lambda i: (i, 0)),
        pl.BlockSpec((BH, 4 * D), lambda i: (i, 0)),
        pl.BlockSpec((BH, 4 * D), lambda i: (i, 0)),
        pl.BlockSpec((BH, 2), lambda i: (i, 0)),
    ] + [full(w) for w in weights]
    return pl.pallas_call(
        _tc_body,
        grid=(NB,),
        in_specs=in_specs,
        out_specs=pl.BlockSpec((2, BH, 1), lambda i: (0, i, 0)),
        out_shape=jax.ShapeDtypeStruct((2, BH2, 1), jnp.float32),
        scratch_shapes=[
            pltpu.VMEM((T, BH, 8 * E), jnp.float32),
            pltpu.VMEM((T, BH, 2 * E), jnp.float32),
        ],
    )(keys3, qw, uw, lens_w, *weights)


def _bdiag(w):
    z = jnp.zeros_like(w)
    return jnp.concatenate([jnp.concatenate([w, z], 1),
                            jnp.concatenate([z, w], 1)], 0)


def _t2(b):
    return jnp.concatenate([b, b]).reshape(1, -1)


def kernel(X, emb_user, emb_item, emb_cate, gru_Wih, gru_Whh, gru_bih,
           gru_bhh, att_W1, att_b1, att_W2, att_b2, att_Wd, att_bd,
           aug_Wih, aug_Whh, aug_bih, aug_bhh, dnn_W1, dnn_b1, dnn_W2,
           dnn_b2, out_W, out_b, pred_bias):
    Xi = X.astype(jnp.int32)
    hiT = Xi[:, 3:3 + T].T
    hcT = Xi[:, 3 + T:3 + 2 * T].T

    def bm(a):
        return a.reshape(T, NB, BH).transpose(1, 0, 2).reshape(-1)

    ki = jnp.concatenate([bm(hiT[:, :BH2]), bm(hcT[:, :BH2]),
                          bm(hiT[:, BH2:]), bm(hcT[:, BH2:])])
    qi = jnp.concatenate([Xi[:BH2, 1], Xi[:BH2, 2],
                          Xi[BH2:, 1], Xi[BH2:, 2]])
    ui = Xi[:, 0]
    lens_w = jnp.stack([Xi[:BH2, 3 + 2 * T], Xi[BH2:, 3 + 2 * T]], axis=1)

    keysw, qw, uw = _sc_gather(emb_user, emb_item, emb_cate, ki, qi, ui)

    aW_q = att_W1[0:E] + att_W1[2 * E:3 * E]
    aW_i = att_W1[E:2 * E] - att_W1[2 * E:3 * E]
    aW_m = att_W1[3 * E:4 * E]
    weights = (
        _bdiag(gru_Wih.T), _bdiag(gru_Whh.T),
        _t2(gru_bih), _t2(gru_bhh),
        _bdiag(aW_q), _bdiag(aW_i), _bdiag(aW_m),
        _t2(att_b1), _bdiag(att_W2), _t2(att_b2),
        _bdiag(att_Wd), _t2(att_bd),
        _bdiag(aug_Wih.T), _bdiag(aug_Whh.T),
        _t2(aug_bih), _t2(aug_bhh),
        dnn_W1, dnn_b1.reshape(1, 256), dnn_W2, dnn_b2.reshape(1, 128),
        out_W, (out_b + pred_bias).reshape(1, 1),
    )
    out2 = _tc_forward(keysw, qw, uw, lens_w, *weights)
    return out2.reshape(B, 1)
